# baseline (device time: 604185 ns/iter reference)
import jax
import jax.numpy as jnp
from jax import lax
from jax.experimental import pallas as pl
from jax.experimental.pallas import tpu as pltpu

N_DEV = 32
B, SQ, D = 2, 256, 768
HL, DH = 8, 64
HD = HL * DH


def kernel(x, Wq, Wo, Wk, Wv):
    def body(x_ref, wq_ref, wo_ref, wk_ref, wv_ref, out_ref,
             comm_ref, send_sems, recv_sems):
        my = lax.axis_index("i")
        left = lax.rem(my + (N_DEV - 1), N_DEV)
        right = lax.rem(my + 1, N_DEV)

        barrier = pltpu.get_barrier_semaphore()
        for nbr in (left, right):
            pl.semaphore_signal(barrier, inc=1, device_id=(nbr,),
                                device_id_type=pl.DeviceIdType.MESH)
        pl.semaphore_wait(barrier, 2)

        wq = wq_ref[:].astype(jnp.bfloat16)
        wk = wk_ref[:].astype(jnp.bfloat16)
        wv = wv_ref[:].astype(jnp.bfloat16)
        wo = wo_ref[:].astype(jnp.bfloat16)
        for b in range(B):
            xb = x_ref[b].astype(jnp.bfloat16)
            q = jnp.dot(xb, wq, preferred_element_type=jnp.float32)
            k = jnp.dot(xb, wk, preferred_element_type=jnp.float32)
            v = jnp.dot(xb, wv, preferred_element_type=jnp.float32)
            heads = []
            for h in range(HL):
                sl = slice(h * DH, (h + 1) * DH)
                qs = q[:, sl].astype(jnp.bfloat16)
                ks = k[:, sl].astype(jnp.bfloat16)
                vs = v[:, sl].astype(jnp.bfloat16)
                s = lax.dot_general(
                    qs, ks, (((1,), (1,)), ((), ())),
                    preferred_element_type=jnp.float32) * 0.125
                m = jnp.max(s, axis=1, keepdims=True)
                p = jnp.exp(s - m)
                l = jnp.sum(p, axis=1, keepdims=True)
                o = jnp.dot(p.astype(jnp.bfloat16), vs,
                            preferred_element_type=jnp.float32) / l
                heads.append(o)
            y = jnp.concatenate(heads, axis=1).astype(jnp.bfloat16)
            out_ref[b] = jnp.dot(y, wo, preferred_element_type=jnp.float32)

        comm_ref[0] = out_ref[:]
        for h in range(N_DEV - 1):
            s_slot = h % 2
            r_slot = (h + 1) % 2
            rdma = pltpu.make_async_remote_copy(
                src_ref=comm_ref.at[s_slot],
                dst_ref=comm_ref.at[r_slot],
                send_sem=send_sems.at[s_slot],
                recv_sem=recv_sems.at[r_slot],
                device_id=(right,),
                device_id_type=pl.DeviceIdType.MESH,
            )
            rdma.start()
            rdma.wait()
            out_ref[:] = out_ref[:] + comm_ref[r_slot]

    return pl.pallas_call(
        body,
        out_shape=jax.ShapeDtypeStruct((B, SQ, D), jnp.float32),
        in_specs=[pl.BlockSpec(memory_space=pltpu.VMEM)] * 5,
        out_specs=pl.BlockSpec(memory_space=pltpu.VMEM),
        scratch_shapes=[
            pltpu.VMEM((2, B, SQ, D), jnp.float32),
            pltpu.SemaphoreType.DMA((2,)),
            pltpu.SemaphoreType.DMA((2,)),
        ],
        compiler_params=pltpu.CompilerParams(collective_id=0),
    )(x, Wq, Wo, Wk, Wv)


# device time: 72811 ns/iter; 8.2980x vs baseline; 8.2980x over previous
import jax
import jax.numpy as jnp
from jax import lax
from jax.experimental import pallas as pl
from jax.experimental.pallas import tpu as pltpu

N_DEV = 32
B, SQ, D = 2, 256, 768
HL, DH = 8, 64
R = B * SQ

BITS = (8, 1, 16, 4, 2)
HALVES = (256, 128, 64, 32, 16)
RS_OFF = (0, 256, 384, 448, 480)


def kernel(x, Wq, Wo, Wk, Wv):
    def body(x_ref, wq_ref, wo_ref, wk_ref, wv_ref, out_ref,
             acc_ref, recv_ref, send_sems, recv_sems):
        my = lax.axis_index("i")
        partners = [jnp.bitwise_xor(my, b) for b in BITS]
        terms = [
            jnp.where(jnp.bitwise_and(my, b) != 0, h, 0)
            for b, h in zip(BITS, HALVES)
        ]

        barrier = pltpu.get_barrier_semaphore()
        for p in partners:
            pl.semaphore_signal(barrier, inc=1, device_id=(p,),
                                device_id_type=pl.DeviceIdType.MESH)
        pl.semaphore_wait(barrier, len(partners))

        wq = wq_ref[:].astype(jnp.bfloat16)
        wk = wk_ref[:].astype(jnp.bfloat16)
        wv = wv_ref[:].astype(jnp.bfloat16)
        wo = wo_ref[:].astype(jnp.bfloat16)
        for b in range(B):
            xb = x_ref[b].astype(jnp.bfloat16)
            q = jnp.dot(xb, wq, preferred_element_type=jnp.float32)
            k = jnp.dot(xb, wk, preferred_element_type=jnp.float32)
            v = jnp.dot(xb, wv, preferred_element_type=jnp.float32)
            heads = []
            for h in range(HL):
                sl = slice(h * DH, (h + 1) * DH)
                qs = q[:, sl].astype(jnp.bfloat16)
                ks = k[:, sl].astype(jnp.bfloat16)
                vs = v[:, sl].astype(jnp.bfloat16)
                s = lax.dot_general(
                    qs, ks, (((1,), (1,)), ((), ())),
                    preferred_element_type=jnp.float32) * 0.125
                m = jnp.max(s, axis=1, keepdims=True)
                p = jnp.exp(s - m)
                l = jnp.sum(p, axis=1, keepdims=True)
                o = jnp.dot(p.astype(jnp.bfloat16), vs,
                            preferred_element_type=jnp.float32) / l
                heads.append(o)
            y = jnp.concatenate(heads, axis=1).astype(jnp.bfloat16)
            acc_ref[b * SQ:(b + 1) * SQ, :] = jnp.dot(
                y, wo, preferred_element_type=jnp.float32)

        base = jnp.int32(0)
        for j in range(5):
            h = HALVES[j]
            send_base = base + (h - terms[j])
            rdma = pltpu.make_async_remote_copy(
                src_ref=acc_ref.at[pl.ds(send_base, h)],
                dst_ref=recv_ref.at[pl.ds(RS_OFF[j], h)],
                send_sem=send_sems.at[j],
                recv_sem=recv_sems.at[j],
                device_id=(partners[j],),
                device_id_type=pl.DeviceIdType.MESH,
            )
            rdma.start()
            rdma.wait()
            base = base + terms[j]
            acc_ref[pl.ds(base, h)] = (
                acc_ref[pl.ds(base, h)] + recv_ref[pl.ds(RS_OFF[j], h)]
            )

        out_ref[pl.ds(base, 16)] = acc_ref[pl.ds(base, 16)]

        for j in range(4, -1, -1):
            h = HALVES[j]
            rdma = pltpu.make_async_remote_copy(
                src_ref=out_ref.at[pl.ds(base, h)],
                dst_ref=out_ref.at[pl.ds(base, h)],
                send_sem=send_sems.at[5 + j],
                recv_sem=recv_sems.at[5 + j],
                device_id=(partners[j],),
                device_id_type=pl.DeviceIdType.MESH,
            )
            rdma.start()
            rdma.wait()
            base = base - terms[j]

    out = pl.pallas_call(
        body,
        out_shape=jax.ShapeDtypeStruct((R, D), jnp.float32),
        in_specs=[pl.BlockSpec(memory_space=pltpu.VMEM)] * 5,
        out_specs=pl.BlockSpec(memory_space=pltpu.VMEM),
        scratch_shapes=[
            pltpu.VMEM((R, D), jnp.float32),
            pltpu.VMEM((R, D), jnp.float32),
            pltpu.SemaphoreType.DMA((10,)),
            pltpu.SemaphoreType.DMA((10,)),
        ],
        compiler_params=pltpu.CompilerParams(collective_id=0),
    )(x, Wq, Wo, Wk, Wv)
    return out.reshape(B, SQ, D)


# device time: 56601 ns/iter; 10.6745x vs baseline; 1.2864x over previous
import jax
import jax.numpy as jnp
from jax import lax
from jax.experimental import pallas as pl
from jax.experimental.pallas import tpu as pltpu

N_DEV = 32
B, SQ, D = 2, 256, 768
HL, DH = 8, 64
R = B * SQ

BITS = (8, 1, 16, 4, 2)
HALVES = (256, 128, 64, 32, 16)
RS_OFF = (0, 256, 384, 448, 480)


def kernel(x, Wq, Wo, Wk, Wv):
    def body(x_ref, wq_ref, wo_ref, wk_ref, wv_ref, out_ref,
             acc_ref, send_ref, recv_ref, ag_ref, send_sems, recv_sems):
        my = lax.axis_index("i")
        partners = [jnp.bitwise_xor(my, b) for b in BITS]
        terms = [
            jnp.where(jnp.bitwise_and(my, b) != 0, h, 0)
            for b, h in zip(BITS, HALVES)
        ]

        barrier = pltpu.get_barrier_semaphore()
        for p in partners:
            pl.semaphore_signal(barrier, inc=1, device_id=(p,),
                                device_id_type=pl.DeviceIdType.MESH)
        pl.semaphore_wait(barrier, len(partners))

        wq = wq_ref[:].astype(jnp.bfloat16)
        wk = wk_ref[:].astype(jnp.bfloat16)
        wv = wv_ref[:].astype(jnp.bfloat16)
        wo = wo_ref[:].astype(jnp.bfloat16)
        for b in range(B):
            xb = x_ref[b].astype(jnp.bfloat16)
            q = jnp.dot(xb, wq, preferred_element_type=jnp.float32)
            k = jnp.dot(xb, wk, preferred_element_type=jnp.float32)
            v = jnp.dot(xb, wv, preferred_element_type=jnp.float32)
            heads = []
            for h in range(HL):
                sl = slice(h * DH, (h + 1) * DH)
                qs = q[:, sl].astype(jnp.bfloat16)
                ks = k[:, sl].astype(jnp.bfloat16)
                vs = v[:, sl].astype(jnp.bfloat16)
                s = lax.dot_general(
                    qs, ks, (((1,), (1,)), ((), ())),
                    preferred_element_type=jnp.float32) * 0.125
                m = jnp.max(s, axis=1, keepdims=True)
                p = jnp.exp(s - m)
                l = jnp.sum(p, axis=1, keepdims=True)
                o = jnp.dot(p.astype(jnp.bfloat16), vs,
                            preferred_element_type=jnp.float32) / l
                heads.append(o)
            y = jnp.concatenate(heads, axis=1).astype(jnp.bfloat16)
            acc_ref[b * SQ:(b + 1) * SQ, :] = jnp.dot(
                y, wo, preferred_element_type=jnp.float32)

        base = jnp.int32(0)
        for j in range(5):
            h = HALVES[j]
            send_base = base + (h - terms[j])
            send_ref[pl.ds(RS_OFF[j], h)] = acc_ref[
                pl.ds(send_base, h)].astype(jnp.bfloat16)
            rdma = pltpu.make_async_remote_copy(
                src_ref=send_ref.at[pl.ds(RS_OFF[j], h)],
                dst_ref=recv_ref.at[pl.ds(RS_OFF[j], h)],
                send_sem=send_sems.at[j],
                recv_sem=recv_sems.at[j],
                device_id=(partners[j],),
                device_id_type=pl.DeviceIdType.MESH,
            )
            rdma.start()
            rdma.wait()
            base = base + terms[j]
            acc_ref[pl.ds(base, h)] = (
                acc_ref[pl.ds(base, h)]
                + recv_ref[pl.ds(RS_OFF[j], h)].astype(jnp.float32)
            )

        ag_ref[pl.ds(base, 16)] = acc_ref[pl.ds(base, 16)].astype(jnp.bfloat16)

        for j in range(4, -1, -1):
            h = HALVES[j]
            rdma = pltpu.make_async_remote_copy(
                src_ref=ag_ref.at[pl.ds(base, h)],
                dst_ref=ag_ref.at[pl.ds(base, h)],
                send_sem=send_sems.at[5 + j],
                recv_sem=recv_sems.at[5 + j],
                device_id=(partners[j],),
                device_id_type=pl.DeviceIdType.MESH,
            )
            rdma.start()
            rdma.wait()
            base = base - terms[j]

        out_ref[:] = ag_ref[:].astype(jnp.float32)

    out = pl.pallas_call(
        body,
        out_shape=jax.ShapeDtypeStruct((R, D), jnp.float32),
        in_specs=[pl.BlockSpec(memory_space=pltpu.VMEM)] * 5,
        out_specs=pl.BlockSpec(memory_space=pltpu.VMEM),
        scratch_shapes=[
            pltpu.VMEM((R, D), jnp.float32),
            pltpu.VMEM((R, D), jnp.bfloat16),
            pltpu.VMEM((R, D), jnp.bfloat16),
            pltpu.VMEM((R, D), jnp.bfloat16),
            pltpu.SemaphoreType.DMA((10,)),
            pltpu.SemaphoreType.DMA((10,)),
        ],
        compiler_params=pltpu.CompilerParams(collective_id=0),
    )(x, Wq, Wo, Wk, Wv)
    return out.reshape(B, SQ, D)


# device time: 52863 ns/iter; 11.4293x vs baseline; 1.0707x over previous
import jax
import jax.numpy as jnp
from jax import lax
from jax.experimental import pallas as pl
from jax.experimental.pallas import tpu as pltpu

N_DEV = 32
B, SQ, D = 2, 256, 768
HL, DH = 8, 64
R = B * SQ

BITS = (8, 1, 16, 4, 2)
HALVES = (256, 128, 64, 32, 16)
RS_OFF = (0, 256, 384, 448, 480)


def kernel(x, Wq, Wo, Wk, Wv):
    def body(x_ref, wq_ref, wo_ref, wk_ref, wv_ref, out_ref,
             acc_ref, send_ref, recv_ref, ag_ref, send_sems, recv_sems):
        my = lax.axis_index("i")
        partners = [jnp.bitwise_xor(my, b) for b in BITS]
        terms = [
            jnp.where(jnp.bitwise_and(my, b) != 0, h, 0)
            for b, h in zip(BITS, HALVES)
        ]

        barrier = pltpu.get_barrier_semaphore()
        for p in partners:
            pl.semaphore_signal(barrier, inc=1, device_id=(p,),
                                device_id_type=pl.DeviceIdType.MESH)
        pl.semaphore_wait(barrier, len(partners))

        wq = wq_ref[:].astype(jnp.bfloat16)
        wk = wk_ref[:].astype(jnp.bfloat16)
        wv = wv_ref[:].astype(jnp.bfloat16)
        wo = wo_ref[:].astype(jnp.bfloat16)

        def compute_batch(b):
            xb = x_ref[b].astype(jnp.bfloat16)
            q = jnp.dot(xb, wq, preferred_element_type=jnp.float32)
            k = jnp.dot(xb, wk, preferred_element_type=jnp.float32)
            v = jnp.dot(xb, wv, preferred_element_type=jnp.float32)
            heads = []
            for h in range(HL):
                sl = slice(h * DH, (h + 1) * DH)
                qs = q[:, sl].astype(jnp.bfloat16)
                ks = k[:, sl].astype(jnp.bfloat16)
                vs = v[:, sl].astype(jnp.bfloat16)
                s = lax.dot_general(
                    qs, ks, (((1,), (1,)), ((), ())),
                    preferred_element_type=jnp.float32) * 0.125
                m = jnp.max(s, axis=1, keepdims=True)
                p = jnp.exp(s - m)
                l = jnp.sum(p, axis=1, keepdims=True)
                o = jnp.dot(p.astype(jnp.bfloat16), vs,
                            preferred_element_type=jnp.float32) / l
                heads.append(o)
            y = jnp.concatenate(heads, axis=1).astype(jnp.bfloat16)
            acc_ref[b * SQ:(b + 1) * SQ, :] = jnp.dot(
                y, wo, preferred_element_type=jnp.float32)

        send_first_b0 = jnp.bitwise_and(my, BITS[0]) != 0

        @pl.when(send_first_b0)
        def _():
            compute_batch(0)
            send_ref[0:SQ] = acc_ref[0:SQ].astype(jnp.bfloat16)

        @pl.when(jnp.logical_not(send_first_b0))
        def _():
            compute_batch(1)
            send_ref[0:SQ] = acc_ref[SQ:R].astype(jnp.bfloat16)

        rdma0 = pltpu.make_async_remote_copy(
            src_ref=send_ref.at[pl.ds(0, SQ)],
            dst_ref=recv_ref.at[pl.ds(0, SQ)],
            send_sem=send_sems.at[0],
            recv_sem=recv_sems.at[0],
            device_id=(partners[0],),
            device_id_type=pl.DeviceIdType.MESH,
        )
        rdma0.start()

        @pl.when(send_first_b0)
        def _():
            compute_batch(1)

        @pl.when(jnp.logical_not(send_first_b0))
        def _():
            compute_batch(0)

        rdma0.wait()
        base = terms[0]
        acc_ref[pl.ds(pl.multiple_of(base, 16), SQ)] = (
            acc_ref[pl.ds(pl.multiple_of(base, 16), SQ)]
            + recv_ref[pl.ds(0, SQ)].astype(jnp.float32)
        )

        for j in range(1, 5):
            h = HALVES[j]
            send_base = base + (h - terms[j])
            send_ref[pl.ds(RS_OFF[j], h)] = acc_ref[
                pl.ds(pl.multiple_of(send_base, 16), h)].astype(jnp.bfloat16)
            rdma = pltpu.make_async_remote_copy(
                src_ref=send_ref.at[pl.ds(RS_OFF[j], h)],
                dst_ref=recv_ref.at[pl.ds(RS_OFF[j], h)],
                send_sem=send_sems.at[j],
                recv_sem=recv_sems.at[j],
                device_id=(partners[j],),
                device_id_type=pl.DeviceIdType.MESH,
            )
            rdma.start()
            rdma.wait()
            base = base + terms[j]
            acc_ref[pl.ds(pl.multiple_of(base, 16), h)] = (
                acc_ref[pl.ds(pl.multiple_of(base, 16), h)]
                + recv_ref[pl.ds(RS_OFF[j], h)].astype(jnp.float32)
            )

        ag_ref[pl.ds(pl.multiple_of(base, 16), 16)] = acc_ref[pl.ds(pl.multiple_of(base, 16), 16)].astype(jnp.bfloat16)
        out_ref[pl.ds(pl.multiple_of(base, 16), 16)] = acc_ref[pl.ds(pl.multiple_of(base, 16), 16)]

        pend_off, pend_h = None, None
        for j in range(4, -1, -1):
            h = HALVES[j]
            rdma = pltpu.make_async_remote_copy(
                src_ref=ag_ref.at[pl.ds(pl.multiple_of(base, 16), h)],
                dst_ref=ag_ref.at[pl.ds(pl.multiple_of(base, 16), h)],
                send_sem=send_sems.at[5 + j],
                recv_sem=recv_sems.at[5 + j],
                device_id=(partners[j],),
                device_id_type=pl.DeviceIdType.MESH,
            )
            rdma.start()
            if pend_off is not None:
                out_ref[pl.ds(pl.multiple_of(pend_off, 16), pend_h)] = ag_ref[
                    pl.ds(pl.multiple_of(pend_off, 16), pend_h)].astype(jnp.float32)
            rdma.wait()
            pend_off = base + h - 2 * terms[j]
            pend_h = h
            base = base - terms[j]
        out_ref[pl.ds(pl.multiple_of(pend_off, 16), pend_h)] = ag_ref[
            pl.ds(pl.multiple_of(pend_off, 16), pend_h)].astype(jnp.float32)

    out = pl.pallas_call(
        body,
        out_shape=jax.ShapeDtypeStruct((R, D), jnp.float32),
        in_specs=[pl.BlockSpec(memory_space=pltpu.VMEM)] * 5,
        out_specs=pl.BlockSpec(memory_space=pltpu.VMEM),
        scratch_shapes=[
            pltpu.VMEM((R, D), jnp.float32),
            pltpu.VMEM((R, D), jnp.bfloat16),
            pltpu.VMEM((R, D), jnp.bfloat16),
            pltpu.VMEM((R, D), jnp.bfloat16),
            pltpu.SemaphoreType.DMA((10,)),
            pltpu.SemaphoreType.DMA((10,)),
        ],
        compiler_params=pltpu.CompilerParams(collective_id=0),
    )(x, Wq, Wo, Wk, Wv)
    return out.reshape(B, SQ, D)


# device time: 38020 ns/iter; 15.8912x vs baseline; 1.3904x over previous
import jax
import jax.numpy as jnp
from jax import lax
from jax.experimental import pallas as pl
from jax.experimental.pallas import tpu as pltpu

N_DEV = 32
B, SQ, D = 2, 256, 768
HL, DH = 8, 64
R = B * SQ

RS0_OFF, RS16_OFF = 0, 256
STAGE_ROWS = 512

MASKS = tuple((c & 3) + (4 if c & 4 else 0) + (16 if c & 8 else 0)
              for c in range(1, 16))

S_RS0 = 0
S_RS16 = 1
S_AG16 = 16
S_AGA = 31
N_SEM = 47


def _mo16(i):
    return pl.multiple_of(i, 16)


def kernel(x, Wq, Wo, Wk, Wv):
    def body(x_ref, wq_ref, wo_ref, wk_ref, wv_ref, out_ref,
             acc_ref, send_ref, recv_ref, ag_ref, send_sems, recv_sems):
        my = lax.axis_index("i")
        q2 = jnp.bitwise_and(my, 3)
        t8 = jnp.where(jnp.bitwise_and(my, 8) != 0, 256, 0)
        t4 = jnp.where(jnp.bitwise_and(my, 4) != 0, 32, 0)

        def rc(bits):
            return jnp.bitwise_xor(my, bits)

        barrier = pltpu.get_barrier_semaphore()
        for b in MASKS + (8,):
            pl.semaphore_signal(barrier, inc=1, device_id=(rc(b),),
                                device_id_type=pl.DeviceIdType.MESH)
        pl.semaphore_wait(barrier, 16)

        wq = (wq_ref[:] * 0.125).astype(jnp.bfloat16)
        wk = wk_ref[:].astype(jnp.bfloat16)
        wv = wv_ref[:].astype(jnp.bfloat16)
        wo = wo_ref[:].astype(jnp.bfloat16)

        def compute_batch(row0):
            row0 = pl.multiple_of(row0, SQ)
            xb = x_ref[pl.ds(row0, SQ)].astype(jnp.bfloat16)
            q = jnp.dot(xb, wq,
                        preferred_element_type=jnp.float32).astype(jnp.bfloat16)
            k = jnp.dot(xb, wk,
                        preferred_element_type=jnp.float32).astype(jnp.bfloat16)
            v = jnp.dot(xb, wv,
                        preferred_element_type=jnp.float32).astype(jnp.bfloat16)
            heads = []
            for h in range(HL):
                sl = slice(h * DH, (h + 1) * DH)
                s = lax.dot_general(
                    q[:, sl], k[:, sl], (((1,), (1,)), ((), ())),
                    preferred_element_type=jnp.float32)
                p = jnp.exp(s)
                l = jnp.sum(p, axis=1, keepdims=True)
                o = jnp.dot(p.astype(jnp.bfloat16), v[:, sl],
                            preferred_element_type=jnp.float32) / l
                heads.append(o)
            y = jnp.concatenate(heads, axis=1).astype(jnp.bfloat16)
            acc_ref[pl.ds(row0, SQ)] = jnp.dot(
                y, wo, preferred_element_type=jnp.float32)

        def remote_copy(src, dst, sem, target):
            return pltpu.make_async_remote_copy(
                src_ref=src, dst_ref=dst,
                send_sem=send_sems.at[sem], recv_sem=recv_sems.at[sem],
                device_id=(target,), device_id_type=pl.DeviceIdType.MESH,
            )

        send_row = 256 - t8
        rdma0 = remote_copy(send_ref.at[pl.ds(RS0_OFF, SQ)],
                            recv_ref.at[pl.ds(RS0_OFF, SQ)], S_RS0, rc(8))

        compute_batch(send_row)
        send_ref[RS0_OFF:RS0_OFF + SQ] = acc_ref[
            pl.ds(pl.multiple_of(send_row, SQ), SQ)].astype(jnp.bfloat16)
        rdma0.start()
        compute_batch(t8)

        rdma0.wait()
        base0 = t8
        blk = (acc_ref[pl.ds(_mo16(base0), SQ)]
               + recv_ref[pl.ds(RS0_OFF, SQ)].astype(jnp.float32))
        acc_ref[pl.ds(_mo16(base0), SQ)] = blk

        g4 = (q2 + 4 * jnp.bitwise_and(jnp.right_shift(my, 2), 1)
              + 8 * jnp.bitwise_and(jnp.right_shift(my, 4), 1))
        send_ref[pl.ds(RS16_OFF, 256)] = blk.astype(jnp.bfloat16)
        rs16 = []
        for c, m in enumerate(MASKS, start=1):
            tgt = jnp.bitwise_xor(g4, c)
            rs16.append(remote_copy(
                send_ref.at[pl.ds(_mo16(RS16_OFF + tgt * 16), 16)],
                recv_ref.at[pl.ds(_mo16(RS16_OFF + g4 * 16), 16)],
                S_RS16 + c - 1, rc(m)))
            rs16[-1].start()
        for r in rs16:
            r.wait()
        myb = base0 + g4 * 16
        contrib = sum(
            recv_ref[pl.ds(_mo16(RS16_OFF + jnp.bitwise_xor(g4, c) * 16), 16)
                     ].astype(jnp.float32)
            for c in range(1, 16))
        own = acc_ref[pl.ds(_mo16(myb), 16)] + contrib

        ag_ref[pl.ds(_mo16(myb), 16)] = own.astype(jnp.bfloat16)
        out_ref[pl.ds(_mo16(myb), 16)] = own

        pieces = [remote_copy(ag_ref.at[pl.ds(_mo16(myb), 16)],
                              ag_ref.at[pl.ds(_mo16(myb), 16)], S_AGA, rc(8))]
        pieces[0].start()

        ag16 = []
        for c, m in enumerate(MASKS, start=1):
            ag16.append(remote_copy(
                ag_ref.at[pl.ds(_mo16(myb), 16)],
                ag_ref.at[pl.ds(_mo16(myb), 16)], S_AG16 + c - 1, rc(m)))
            ag16[-1].start()
        for c, m in enumerate(MASKS, start=1):
            ag16[c - 1].wait()
            sb = base0 + jnp.bitwise_xor(g4, c) * 16
            p = remote_copy(ag_ref.at[pl.ds(_mo16(sb), 16)],
                            ag_ref.at[pl.ds(_mo16(sb), 16)],
                            S_AGA + c, rc(8))
            p.start()
            pieces.append(p)
        out_ref[pl.ds(_mo16(base0), SQ)] = ag_ref[
            pl.ds(_mo16(base0), SQ)].astype(jnp.float32)
        for p in pieces:
            p.wait()
        sib_a = 256 - t8
        out_ref[pl.ds(_mo16(sib_a), SQ)] = ag_ref[
            pl.ds(_mo16(sib_a), SQ)].astype(jnp.float32)

    out = pl.pallas_call(
        body,
        out_shape=jax.ShapeDtypeStruct((R, D), jnp.float32),
        in_specs=[pl.BlockSpec(memory_space=pltpu.VMEM)] * 5,
        out_specs=pl.BlockSpec(memory_space=pltpu.VMEM),
        scratch_shapes=[
            pltpu.VMEM((R, D), jnp.float32),
            pltpu.VMEM((STAGE_ROWS, D), jnp.bfloat16),
            pltpu.VMEM((STAGE_ROWS, D), jnp.bfloat16),
            pltpu.VMEM((R, D), jnp.bfloat16),
            pltpu.SemaphoreType.DMA((N_SEM,)),
            pltpu.SemaphoreType.DMA((N_SEM,)),
        ],
        compiler_params=pltpu.CompilerParams(collective_id=0),
    )(x.reshape(R, D), Wq, Wo, Wk, Wv)
    return out.reshape(B, SQ, D)
